# baseline (speedup 1.0000x reference)
import jax
import jax.numpy as jnp
from jax.experimental import pallas as pl

S = 2048
DIM = 768
H = 12
DH = 64
BLK = 4
NB = S // BLK
WIN = 32
QC = 256
NQ = S // QC
GCOL = 128
SCALE = DH ** -0.5
F32 = jnp.float32
BF16 = jnp.bfloat16


def _prep_kernel(x_ref, x0_ref, lam_ref, w_ref, x1_ref, y_ref):
    lam0 = lam_ref[0, 0]
    lam1 = lam_ref[0, 1]
    x1 = lam0 * x_ref[...] + lam1 * x0_ref[...]
    x1_ref[...] = x1
    h = x1 * jax.lax.rsqrt(jnp.mean(x1 * x1, axis=-1, keepdims=True) + 1e-6)
    y = jnp.dot(h.astype(BF16), w_ref[...], preferred_element_type=F32)
    y_ref[:, : 3 * DIM] = y[:, : 3 * DIM]
    y_ref[:, 3 * DIM :] = jax.nn.sigmoid(y[:, 3 * DIM :])


def _ckv_kernel(kh_ref, vh_ref, wkc_ref, wvc_ref, kpe_ref, vpe_ref, ck_ref, cv_ref):
    pe_k = jnp.dot(kpe_ref[...], wkc_ref[...], preferred_element_type=F32)
    pe_v = jnp.dot(vpe_ref[...], wvc_ref[...], preferred_element_type=F32)
    ck_ref[0] = (jnp.dot(kh_ref[0], wkc_ref[...], preferred_element_type=F32) + pe_k).astype(BF16)
    cv_ref[0] = (jnp.dot(vh_ref[0], wvc_ref[...], preferred_element_type=F32) + pe_v).astype(BF16)


KC = 256
CBC = KC // BLK


def _attn_kernel(q_ref, k_ref, v_ref, ck_ref, cv_ref, fm_ref, g_ref, out_ref):
    qc = pl.program_id(1)
    q = q_ref[0]
    rowf = qc * QC + jax.lax.broadcasted_iota(jnp.int32, (QC, KC), 0)
    rowb = qc * QC + jax.lax.broadcasted_iota(jnp.int32, (QC, CBC), 0)
    colb = jax.lax.broadcasted_iota(jnp.int32, (QC, CBC), 1)

    mf = jnp.full((QC, 1), -1e30, F32)
    lf = jnp.zeros((QC, 1), F32)
    of = jnp.zeros((QC, DH), F32)
    mc = jnp.zeros((QC, 1), F32)
    lc = jnp.ones((QC, 1), F32)
    oc = jnp.zeros((QC, DH), F32)

    def body(i, carry):
        mf, lf, of, mc, lc, oc = carry
        kblk = k_ref[0, pl.ds(i * KC, KC), :]
        vblk = v_ref[0, pl.ds(i * KC, KC), :]
        sim = jax.lax.dot_general(q, kblk, (((1,), (1,)), ((), ())),
                                  preferred_element_type=F32) * SCALE
        fmb = fm_ref[:, pl.ds(i * KC, KC)].astype(F32)
        sf = sim * fmb + (fmb - 1.0) * 1e9
        mf_n = jnp.maximum(mf, jnp.max(sf, axis=-1, keepdims=True))
        af = jnp.exp(mf - mf_n)
        pf = jnp.exp(sf - mf_n) * fmb
        lf = lf * af + jnp.sum(pf, axis=-1, keepdims=True)
        of = of * af + jax.lax.dot_general(pf.astype(BF16), vblk,
                                           (((1,), (0,)), ((), ())),
                                           preferred_element_type=F32)
        ckb = ck_ref[0, pl.ds(i * CBC, CBC), :]
        cvb = cv_ref[0, pl.ds(i * CBC, CBC), :]
        simc = jax.lax.dot_general(q, ckb, (((1,), (1,)), ((), ())),
                                   preferred_element_type=F32) * SCALE
        cm = ((i * CBC + colb + 1) * BLK - 1) <= rowb
        sc = jnp.where(cm, simc, -1e9)
        mc_n = jnp.maximum(mc, jnp.max(sc, axis=-1, keepdims=True))
        ac = jnp.exp(mc - mc_n)
        pc = jnp.where(cm, jnp.exp(sc - mc_n), 0.0)
        lc = lc * ac + jnp.sum(pc, axis=-1, keepdims=True)
        oc = oc * ac + jax.lax.dot_general(pc.astype(BF16), cvb,
                                           (((1,), (0,)), ((), ())),
                                           preferred_element_type=F32)
        return mf_n, lf, of, mc_n, lc, oc

    mf, lf, of, mc, lc, oc = jax.lax.fori_loop(
        0, qc + 1, body, (mf, lf, of, mc, lc, oc))
    f_out = of / lf
    c_out = oc / lc

    start = jnp.maximum(qc - 1, 0) * KC
    kb = k_ref[0, pl.ds(start, 2 * KC), :]
    vb = v_ref[0, pl.ds(start, 2 * KC), :]
    sims = jax.lax.dot_general(q, kb, (((1,), (1,)), ((), ())),
                               preferred_element_type=F32) * SCALE
    rows = qc * QC + jax.lax.broadcasted_iota(jnp.int32, (QC, 2 * KC), 0)
    cols = start + jax.lax.broadcasted_iota(jnp.int32, (QC, 2 * KC), 1)
    sm = (cols <= rows) & ((rows - cols) < WIN)
    ss = jnp.where(sm, sims, -1e9)
    ms = jnp.max(ss, axis=-1, keepdims=True)
    ps = jnp.where(sm, jnp.exp(ss - ms), 0.0)
    s_out = jax.lax.dot_general(ps.astype(BF16), vb, (((1,), (0,)), ((), ())),
                                preferred_element_type=F32) / jnp.sum(
        ps, axis=-1, keepdims=True
    )

    g = g_ref[0]
    out_ref[0] = g[:, 0:1] * c_out + g[:, 1:2] * f_out + g[:, 2:3] * s_out


def _mlp_kernel(attn_ref, x1_ref, wo_ref, wfc_ref, wproj_ref, y_ref):
    x2 = x1_ref[...] + jnp.dot(attn_ref[...], wo_ref[...], preferred_element_type=F32)
    h2 = x2 * jax.lax.rsqrt(jnp.mean(x2 * x2, axis=-1, keepdims=True) + 1e-6)
    u = jnp.dot(h2.astype(BF16), wfc_ref[...], preferred_element_type=F32)
    u = jnp.square(jnp.maximum(u, 0.0))
    y_ref[...] = x2 + jnp.dot(u.astype(BF16), wproj_ref[...], preferred_element_type=F32)


def kernel(x, ve, x0, lambdas, Wq, Wk, Wv, Wo, k_pe, v_pe, Wkc, Wvc, Wg, Wfc, Wproj,
           sliding_window_flex_mask, fine_selection_flex_mask):
    del ve, sliding_window_flex_mask
    x2d = x[0]
    x02d = x0[0]
    w_all = jnp.concatenate(
        [Wq, Wk, Wv, jnp.pad(Wg, ((0, 0), (0, GCOL - 3 * H)))], axis=1
    ).astype(BF16)
    lam2 = lambdas.reshape(1, 2)

    x1, y = pl.pallas_call(
        _prep_kernel,
        grid=(NQ,),
        in_specs=[
            pl.BlockSpec((QC, DIM), lambda i: (i, 0)),
            pl.BlockSpec((QC, DIM), lambda i: (i, 0)),
            pl.BlockSpec((1, 2), lambda i: (0, 0)),
            pl.BlockSpec((DIM, 3 * DIM + GCOL), lambda i: (0, 0)),
        ],
        out_specs=[
            pl.BlockSpec((QC, DIM), lambda i: (i, 0)),
            pl.BlockSpec((QC, 3 * DIM + GCOL), lambda i: (i, 0)),
        ],
        out_shape=[
            jax.ShapeDtypeStruct((S, DIM), F32),
            jax.ShapeDtypeStruct((S, 3 * DIM + GCOL), F32),
        ],
    )(x2d, x02d, lam2, w_all)

    qkv = y[:, : 3 * DIM].reshape(S, 3, H, DH).transpose(1, 2, 0, 3)
    qkv16 = qkv.astype(BF16)
    q_hm = qkv16[0]
    k_hm = qkv16[1]
    v_hm = qkv16[2]
    g_hm = y[:, 3 * DIM : 3 * DIM + 3 * H].reshape(S, H, 3).transpose(1, 0, 2)
    kh = k_hm.reshape(H, NB, BLK * DH)
    vh = v_hm.reshape(H, NB, BLK * DH)

    ck, cv = pl.pallas_call(
        _ckv_kernel,
        grid=(H,),
        in_specs=[
            pl.BlockSpec((1, NB, BLK * DH), lambda h: (h, 0, 0)),
            pl.BlockSpec((1, NB, BLK * DH), lambda h: (h, 0, 0)),
            pl.BlockSpec((BLK * DH, DH), lambda h: (0, 0)),
            pl.BlockSpec((BLK * DH, DH), lambda h: (0, 0)),
            pl.BlockSpec((1, BLK * DH), lambda h: (0, 0)),
            pl.BlockSpec((1, BLK * DH), lambda h: (0, 0)),
        ],
        out_specs=[
            pl.BlockSpec((1, NB, DH), lambda h: (h, 0, 0)),
            pl.BlockSpec((1, NB, DH), lambda h: (h, 0, 0)),
        ],
        out_shape=[
            jax.ShapeDtypeStruct((H, NB, DH), BF16),
            jax.ShapeDtypeStruct((H, NB, DH), BF16),
        ],
    )(kh, vh, Wkc.astype(BF16), Wvc.astype(BF16),
      k_pe.reshape(1, BLK * DH).astype(BF16), v_pe.reshape(1, BLK * DH).astype(BF16))

    fm8 = fine_selection_flex_mask.astype(jnp.int8)

    attn = pl.pallas_call(
        _attn_kernel,
        grid=(H, NQ),
        in_specs=[
            pl.BlockSpec((1, QC, DH), lambda h, i: (h, i, 0)),
            pl.BlockSpec((1, S, DH), lambda h, i: (h, 0, 0)),
            pl.BlockSpec((1, S, DH), lambda h, i: (h, 0, 0)),
            pl.BlockSpec((1, NB, DH), lambda h, i: (h, 0, 0)),
            pl.BlockSpec((1, NB, DH), lambda h, i: (h, 0, 0)),
            pl.BlockSpec((QC, S), lambda h, i: (i, 0)),
            pl.BlockSpec((1, QC, 3), lambda h, i: (h, i, 0)),
        ],
        out_specs=pl.BlockSpec((1, QC, DH), lambda h, i: (h, i, 0)),
        out_shape=jax.ShapeDtypeStruct((H, S, DH), F32),
    )(q_hm, k_hm, v_hm, ck, cv, fm8, g_hm)

    attn2d = attn.transpose(1, 0, 2).reshape(S, H * DH).astype(BF16)

    out = pl.pallas_call(
        _mlp_kernel,
        grid=(NQ,),
        in_specs=[
            pl.BlockSpec((QC, H * DH), lambda i: (i, 0)),
            pl.BlockSpec((QC, DIM), lambda i: (i, 0)),
            pl.BlockSpec((H * DH, DIM), lambda i: (0, 0)),
            pl.BlockSpec((DIM, 4 * DIM), lambda i: (0, 0)),
            pl.BlockSpec((4 * DIM, DIM), lambda i: (0, 0)),
        ],
        out_specs=pl.BlockSpec((QC, DIM), lambda i: (i, 0)),
        out_shape=jax.ShapeDtypeStruct((S, DIM), F32),
    )(attn2d, x1, Wo.astype(BF16), Wfc.astype(BF16), Wproj.astype(BF16))

    return out[None]


# speedup vs baseline: 1.5031x; 1.5031x over previous
import jax
import jax.numpy as jnp
from jax.experimental import pallas as pl

S = 2048
DIM = 768
H = 12
DH = 64
BLK = 4
NB = S // BLK
WIN = 32
QC = 256
NQ = S // QC
GCOL = 128
SCALE = DH ** -0.5
F32 = jnp.float32
BF16 = jnp.bfloat16


def _prep_kernel(x_ref, x0_ref, lam_ref, w_ref, x1_ref, y_ref):
    lam0 = lam_ref[0, 0]
    lam1 = lam_ref[0, 1]
    x1 = lam0 * x_ref[...] + lam1 * x0_ref[...]
    x1_ref[...] = x1
    h = x1 * jax.lax.rsqrt(jnp.mean(x1 * x1, axis=-1, keepdims=True) + 1e-6)
    y = jnp.dot(h.astype(BF16), w_ref[...], preferred_element_type=F32)
    y_ref[:, : 3 * DIM] = y[:, : 3 * DIM]
    y_ref[:, 3 * DIM :] = jax.nn.sigmoid(y[:, 3 * DIM :])


def _ckv_kernel(kh_ref, vh_ref, wkc_ref, wvc_ref, kpe_ref, vpe_ref, ck_ref, cv_ref):
    pe_k = jnp.dot(kpe_ref[...], wkc_ref[...], preferred_element_type=F32)
    pe_v = jnp.dot(vpe_ref[...], wvc_ref[...], preferred_element_type=F32)
    ck_ref[0] = (jnp.dot(kh_ref[0], wkc_ref[...], preferred_element_type=F32) + pe_k).astype(BF16)
    cv_ref[0] = (jnp.dot(vh_ref[0], wvc_ref[...], preferred_element_type=F32) + pe_v).astype(BF16)


KC = 256
CBC = KC // BLK


def _attn_kernel(q_ref, k_ref, v_ref, ck_ref, cv_ref, fm_ref, g_ref, out_ref):
    qc = pl.program_id(1)
    q = q_ref[0]
    k = k_ref[0]
    v = v_ref[0]
    sim = jax.lax.dot_general(q, k, (((1,), (1,)), ((), ())),
                              preferred_element_type=F32) * SCALE

    fm = fm_ref[...].astype(F32)
    sf = sim * fm + (fm - 1.0) * 1e9
    mf = jnp.max(sf, axis=-1, keepdims=True)
    pf = jnp.exp(sf - mf)
    f_out = jax.lax.dot_general(pf.astype(BF16), v, (((1,), (0,)), ((), ())),
                                preferred_element_type=F32) / jnp.sum(
        pf, axis=-1, keepdims=True
    )

    ck = ck_ref[0]
    cv = cv_ref[0]
    simc = jax.lax.dot_general(q, ck, (((1,), (1,)), ((), ())),
                               preferred_element_type=F32) * SCALE
    rowc = qc * QC + jax.lax.broadcasted_iota(jnp.int32, (QC, NB), 0)
    colc = jax.lax.broadcasted_iota(jnp.int32, (QC, NB), 1)
    cmask = ((colc + 1) * BLK - 1) <= rowc
    sc = jnp.where(cmask, simc, -1e9)
    mc = jnp.maximum(jnp.max(sc, axis=-1, keepdims=True), 0.0)
    pc = jnp.where(cmask, jnp.exp(sc - mc), 0.0)
    den = jnp.sum(pc, axis=-1, keepdims=True) + jnp.exp(-mc)
    c_out = jax.lax.dot_general(pc.astype(BF16), cv, (((1,), (0,)), ((), ())),
                                preferred_element_type=F32) / den

    start = jnp.maximum(qc - 1, 0) * KC
    kb = k_ref[0, pl.ds(start, 2 * KC), :]
    vb = v_ref[0, pl.ds(start, 2 * KC), :]
    sims = jax.lax.dot_general(q, kb, (((1,), (1,)), ((), ())),
                               preferred_element_type=F32) * SCALE
    rows = qc * QC + jax.lax.broadcasted_iota(jnp.int32, (QC, 2 * KC), 0)
    cols = start + jax.lax.broadcasted_iota(jnp.int32, (QC, 2 * KC), 1)
    sm = (cols <= rows) & ((rows - cols) < WIN)
    ss = jnp.where(sm, sims, -1e9)
    ms = jnp.max(ss, axis=-1, keepdims=True)
    ps = jnp.where(sm, jnp.exp(ss - ms), 0.0)
    s_out = jax.lax.dot_general(ps.astype(BF16), vb, (((1,), (0,)), ((), ())),
                                preferred_element_type=F32) / jnp.sum(
        ps, axis=-1, keepdims=True
    )

    g = g_ref[0]
    out_ref[0] = g[:, 0:1] * c_out + g[:, 1:2] * f_out + g[:, 2:3] * s_out


def _mlp_kernel(attn_ref, x1_ref, wo_ref, wfc_ref, wproj_ref, y_ref):
    x2 = x1_ref[...] + jnp.dot(attn_ref[...], wo_ref[...], preferred_element_type=F32)
    h2 = x2 * jax.lax.rsqrt(jnp.mean(x2 * x2, axis=-1, keepdims=True) + 1e-6)
    u = jnp.dot(h2.astype(BF16), wfc_ref[...], preferred_element_type=F32)
    u = jnp.square(jnp.maximum(u, 0.0))
    y_ref[...] = x2 + jnp.dot(u.astype(BF16), wproj_ref[...], preferred_element_type=F32)


def kernel(x, ve, x0, lambdas, Wq, Wk, Wv, Wo, k_pe, v_pe, Wkc, Wvc, Wg, Wfc, Wproj,
           sliding_window_flex_mask, fine_selection_flex_mask):
    del ve, sliding_window_flex_mask
    x2d = x[0]
    x02d = x0[0]
    w_all = jnp.concatenate(
        [Wq, Wk, Wv, jnp.pad(Wg, ((0, 0), (0, GCOL - 3 * H)))], axis=1
    ).astype(BF16)
    lam2 = lambdas.reshape(1, 2)

    x1, y = pl.pallas_call(
        _prep_kernel,
        grid=(NQ,),
        in_specs=[
            pl.BlockSpec((QC, DIM), lambda i: (i, 0)),
            pl.BlockSpec((QC, DIM), lambda i: (i, 0)),
            pl.BlockSpec((1, 2), lambda i: (0, 0)),
            pl.BlockSpec((DIM, 3 * DIM + GCOL), lambda i: (0, 0)),
        ],
        out_specs=[
            pl.BlockSpec((QC, DIM), lambda i: (i, 0)),
            pl.BlockSpec((QC, 3 * DIM + GCOL), lambda i: (i, 0)),
        ],
        out_shape=[
            jax.ShapeDtypeStruct((S, DIM), F32),
            jax.ShapeDtypeStruct((S, 3 * DIM + GCOL), F32),
        ],
    )(x2d, x02d, lam2, w_all)

    qkv = y[:, : 3 * DIM].reshape(S, 3, H, DH).transpose(1, 2, 0, 3)
    qkv16 = qkv.astype(BF16)
    q_hm = qkv16[0]
    k_hm = qkv16[1]
    v_hm = qkv16[2]
    g_hm = y[:, 3 * DIM : 3 * DIM + 3 * H].reshape(S, H, 3).transpose(1, 0, 2)
    kh = k_hm.reshape(H, NB, BLK * DH)
    vh = v_hm.reshape(H, NB, BLK * DH)

    ck, cv = pl.pallas_call(
        _ckv_kernel,
        grid=(H,),
        in_specs=[
            pl.BlockSpec((1, NB, BLK * DH), lambda h: (h, 0, 0)),
            pl.BlockSpec((1, NB, BLK * DH), lambda h: (h, 0, 0)),
            pl.BlockSpec((BLK * DH, DH), lambda h: (0, 0)),
            pl.BlockSpec((BLK * DH, DH), lambda h: (0, 0)),
            pl.BlockSpec((1, BLK * DH), lambda h: (0, 0)),
            pl.BlockSpec((1, BLK * DH), lambda h: (0, 0)),
        ],
        out_specs=[
            pl.BlockSpec((1, NB, DH), lambda h: (h, 0, 0)),
            pl.BlockSpec((1, NB, DH), lambda h: (h, 0, 0)),
        ],
        out_shape=[
            jax.ShapeDtypeStruct((H, NB, DH), BF16),
            jax.ShapeDtypeStruct((H, NB, DH), BF16),
        ],
    )(kh, vh, Wkc.astype(BF16), Wvc.astype(BF16),
      k_pe.reshape(1, BLK * DH).astype(BF16), v_pe.reshape(1, BLK * DH).astype(BF16))

    fm8 = fine_selection_flex_mask.astype(jnp.int8)

    attn = pl.pallas_call(
        _attn_kernel,
        grid=(H, NQ),
        in_specs=[
            pl.BlockSpec((1, QC, DH), lambda h, i: (h, i, 0)),
            pl.BlockSpec((1, S, DH), lambda h, i: (h, 0, 0)),
            pl.BlockSpec((1, S, DH), lambda h, i: (h, 0, 0)),
            pl.BlockSpec((1, NB, DH), lambda h, i: (h, 0, 0)),
            pl.BlockSpec((1, NB, DH), lambda h, i: (h, 0, 0)),
            pl.BlockSpec((QC, S), lambda h, i: (i, 0)),
            pl.BlockSpec((1, QC, 3), lambda h, i: (h, i, 0)),
        ],
        out_specs=pl.BlockSpec((1, QC, DH), lambda h, i: (h, i, 0)),
        out_shape=jax.ShapeDtypeStruct((H, S, DH), F32),
    )(q_hm, k_hm, v_hm, ck, cv, fm8, g_hm)

    attn2d = attn.transpose(1, 0, 2).reshape(S, H * DH).astype(BF16)

    out = pl.pallas_call(
        _mlp_kernel,
        grid=(NQ,),
        in_specs=[
            pl.BlockSpec((QC, H * DH), lambda i: (i, 0)),
            pl.BlockSpec((QC, DIM), lambda i: (i, 0)),
            pl.BlockSpec((H * DH, DIM), lambda i: (0, 0)),
            pl.BlockSpec((DIM, 4 * DIM), lambda i: (0, 0)),
            pl.BlockSpec((4 * DIM, DIM), lambda i: (0, 0)),
        ],
        out_specs=pl.BlockSpec((QC, DIM), lambda i: (i, 0)),
        out_shape=jax.ShapeDtypeStruct((S, DIM), F32),
    )(attn2d, x1, Wo.astype(BF16), Wfc.astype(BF16), Wproj.astype(BF16))

    return out[None]


# speedup vs baseline: 1.8701x; 1.2442x over previous
import jax
import jax.numpy as jnp
from jax.experimental import pallas as pl

S = 2048
DIM = 768
H = 12
DH = 64
BLK = 4
NB = S // BLK
WIN = 32
QC = 256
NQ = S // QC
GCOL = 128
SCALE = DH ** -0.5
F32 = jnp.float32
BF16 = jnp.bfloat16


def _prep_kernel(x_ref, x0_ref, lam_ref, w_ref, x1_ref, y_ref):
    lam0 = lam_ref[0, 0]
    lam1 = lam_ref[0, 1]
    x1 = lam0 * x_ref[...] + lam1 * x0_ref[...]
    x1_ref[...] = x1
    h = x1 * jax.lax.rsqrt(jnp.mean(x1 * x1, axis=-1, keepdims=True) + 1e-6)
    y = jnp.dot(h.astype(BF16), w_ref[...], preferred_element_type=F32)
    y_ref[:, : 3 * DIM] = y[:, : 3 * DIM]
    y_ref[:, 3 * DIM :] = jax.nn.sigmoid(y[:, 3 * DIM :])


def _ckv_kernel(kh_ref, vh_ref, wkc_ref, wvc_ref, kpe_ref, vpe_ref, ck_ref, cv_ref):
    pe_k = jnp.dot(kpe_ref[...], wkc_ref[...], preferred_element_type=F32)
    pe_v = jnp.dot(vpe_ref[...], wvc_ref[...], preferred_element_type=F32)
    ck_ref[0] = (jnp.dot(kh_ref[0], wkc_ref[...], preferred_element_type=F32) + pe_k).astype(BF16)
    cv_ref[0] = (jnp.dot(vh_ref[0], wvc_ref[...], preferred_element_type=F32) + pe_v).astype(BF16)


QR = 512
NCALL = S // QR
SB = QR + 64


def _attn_band_kernel(j, q_ref, k_ref, v_ref, ck_ref, cv_ref, fm_ref, g_ref,
                      out_ref):
    W = (j + 1) * QR
    CB = W // BLK
    s0 = max(0, j * QR - 64)
    q = q_ref[0]
    k = k_ref[0]
    v = v_ref[0]
    sim = jax.lax.dot_general(q, k, (((1,), (1,)), ((), ())),
                              preferred_element_type=F32) * SCALE

    fm = fm_ref[...].astype(F32)
    sf = sim * fm + (fm - 1.0) * 1e9
    mf = jnp.max(sf, axis=-1, keepdims=True)
    pf = jnp.exp(sf - mf)
    f_out = jax.lax.dot_general(pf.astype(BF16), v, (((1,), (0,)), ((), ())),
                                preferred_element_type=F32) / jnp.sum(
        pf, axis=-1, keepdims=True
    )

    ck = ck_ref[0]
    cv = cv_ref[0]
    simc = jax.lax.dot_general(q, ck, (((1,), (1,)), ((), ())),
                               preferred_element_type=F32) * SCALE
    rowc = j * QR + jax.lax.broadcasted_iota(jnp.int32, (QR, CB), 0)
    colc = jax.lax.broadcasted_iota(jnp.int32, (QR, CB), 1)
    cmask = ((colc + 1) * BLK - 1) <= rowc
    sc = jnp.where(cmask, simc, -1e9)
    mc = jnp.maximum(jnp.max(sc, axis=-1, keepdims=True), 0.0)
    pc = jnp.where(cmask, jnp.exp(sc - mc), 0.0)
    den = jnp.sum(pc, axis=-1, keepdims=True) + jnp.exp(-mc)
    c_out = jax.lax.dot_general(pc.astype(BF16), cv, (((1,), (0,)), ((), ())),
                                preferred_element_type=F32) / den

    sbw = min(SB, W - s0)
    kb = k[s0 : s0 + sbw, :]
    vb = v[s0 : s0 + sbw, :]
    sims = jax.lax.dot_general(q, kb, (((1,), (1,)), ((), ())),
                               preferred_element_type=F32) * SCALE
    rows = j * QR + jax.lax.broadcasted_iota(jnp.int32, (QR, sbw), 0)
    cols = s0 + jax.lax.broadcasted_iota(jnp.int32, (QR, sbw), 1)
    sm = (cols <= rows) & ((rows - cols) < WIN)
    ss = jnp.where(sm, sims, -1e9)
    ms = jnp.max(ss, axis=-1, keepdims=True)
    ps = jnp.where(sm, jnp.exp(ss - ms), 0.0)
    s_out = jax.lax.dot_general(ps.astype(BF16), vb, (((1,), (0,)), ((), ())),
                                preferred_element_type=F32) / jnp.sum(
        ps, axis=-1, keepdims=True
    )

    g = g_ref[0]
    out_ref[0] = g[:, 0:1] * c_out + g[:, 1:2] * f_out + g[:, 2:3] * s_out


def _mlp_kernel(attn_ref, x1_ref, wo_ref, wfc_ref, wproj_ref, y_ref):
    x2 = x1_ref[...] + jnp.dot(attn_ref[...], wo_ref[...], preferred_element_type=F32)
    h2 = x2 * jax.lax.rsqrt(jnp.mean(x2 * x2, axis=-1, keepdims=True) + 1e-6)
    u = jnp.dot(h2.astype(BF16), wfc_ref[...], preferred_element_type=F32)
    u = jnp.square(jnp.maximum(u, 0.0))
    y_ref[...] = x2 + jnp.dot(u.astype(BF16), wproj_ref[...], preferred_element_type=F32)


def kernel(x, ve, x0, lambdas, Wq, Wk, Wv, Wo, k_pe, v_pe, Wkc, Wvc, Wg, Wfc, Wproj,
           sliding_window_flex_mask, fine_selection_flex_mask):
    del ve, sliding_window_flex_mask
    x2d = x[0]
    x02d = x0[0]
    w_all = jnp.concatenate(
        [Wq, Wk, Wv, jnp.pad(Wg, ((0, 0), (0, GCOL - 3 * H)))], axis=1
    ).astype(BF16)
    lam2 = lambdas.reshape(1, 2)

    x1, y = pl.pallas_call(
        _prep_kernel,
        grid=(NQ,),
        in_specs=[
            pl.BlockSpec((QC, DIM), lambda i: (i, 0)),
            pl.BlockSpec((QC, DIM), lambda i: (i, 0)),
            pl.BlockSpec((1, 2), lambda i: (0, 0)),
            pl.BlockSpec((DIM, 3 * DIM + GCOL), lambda i: (0, 0)),
        ],
        out_specs=[
            pl.BlockSpec((QC, DIM), lambda i: (i, 0)),
            pl.BlockSpec((QC, 3 * DIM + GCOL), lambda i: (i, 0)),
        ],
        out_shape=[
            jax.ShapeDtypeStruct((S, DIM), F32),
            jax.ShapeDtypeStruct((S, 3 * DIM + GCOL), F32),
        ],
    )(x2d, x02d, lam2, w_all)

    qkv = y[:, : 3 * DIM].reshape(S, 3, H, DH).transpose(1, 2, 0, 3)
    qkv16 = qkv.astype(BF16)
    q_hm = qkv16[0]
    k_hm = qkv16[1]
    v_hm = qkv16[2]
    g_hm = y[:, 3 * DIM : 3 * DIM + 3 * H].reshape(S, H, 3).transpose(1, 0, 2)
    kh = k_hm.reshape(H, NB, BLK * DH)
    vh = v_hm.reshape(H, NB, BLK * DH)

    ck, cv = pl.pallas_call(
        _ckv_kernel,
        grid=(H,),
        in_specs=[
            pl.BlockSpec((1, NB, BLK * DH), lambda h: (h, 0, 0)),
            pl.BlockSpec((1, NB, BLK * DH), lambda h: (h, 0, 0)),
            pl.BlockSpec((BLK * DH, DH), lambda h: (0, 0)),
            pl.BlockSpec((BLK * DH, DH), lambda h: (0, 0)),
            pl.BlockSpec((1, BLK * DH), lambda h: (0, 0)),
            pl.BlockSpec((1, BLK * DH), lambda h: (0, 0)),
        ],
        out_specs=[
            pl.BlockSpec((1, NB, DH), lambda h: (h, 0, 0)),
            pl.BlockSpec((1, NB, DH), lambda h: (h, 0, 0)),
        ],
        out_shape=[
            jax.ShapeDtypeStruct((H, NB, DH), BF16),
            jax.ShapeDtypeStruct((H, NB, DH), BF16),
        ],
    )(kh, vh, Wkc.astype(BF16), Wvc.astype(BF16),
      k_pe.reshape(1, BLK * DH).astype(BF16), v_pe.reshape(1, BLK * DH).astype(BF16))

    fm8 = fine_selection_flex_mask.astype(jnp.int8)

    import functools
    attn_parts = []
    for j in range(NCALL):
        W = (j + 1) * QR
        CB = W // BLK
        attn_parts.append(pl.pallas_call(
            functools.partial(_attn_band_kernel, j),
            grid=(H,),
            in_specs=[
                pl.BlockSpec((1, QR, DH), lambda h, j=j: (h, j, 0)),
                pl.BlockSpec((1, W, DH), lambda h: (h, 0, 0)),
                pl.BlockSpec((1, W, DH), lambda h: (h, 0, 0)),
                pl.BlockSpec((1, CB, DH), lambda h: (h, 0, 0)),
                pl.BlockSpec((1, CB, DH), lambda h: (h, 0, 0)),
                pl.BlockSpec((QR, W), lambda h, j=j: (j, 0)),
                pl.BlockSpec((1, QR, 3), lambda h, j=j: (h, j, 0)),
            ],
            out_specs=pl.BlockSpec((1, QR, DH), lambda h: (h, 0, 0)),
            out_shape=jax.ShapeDtypeStruct((H, QR, DH), F32),
        )(q_hm, k_hm, v_hm, ck, cv, fm8, g_hm))

    attn = jnp.concatenate(attn_parts, axis=1)
    attn2d = attn.transpose(1, 0, 2).reshape(S, H * DH).astype(BF16)

    out = pl.pallas_call(
        _mlp_kernel,
        grid=(NQ,),
        in_specs=[
            pl.BlockSpec((QC, H * DH), lambda i: (i, 0)),
            pl.BlockSpec((QC, DIM), lambda i: (i, 0)),
            pl.BlockSpec((H * DH, DIM), lambda i: (0, 0)),
            pl.BlockSpec((DIM, 4 * DIM), lambda i: (0, 0)),
            pl.BlockSpec((4 * DIM, DIM), lambda i: (0, 0)),
        ],
        out_specs=pl.BlockSpec((QC, DIM), lambda i: (i, 0)),
        out_shape=jax.ShapeDtypeStruct((S, DIM), F32),
    )(attn2d, x1, Wo.astype(BF16), Wfc.astype(BF16), Wproj.astype(BF16))

    return out[None]


# speedup vs baseline: 2.0211x; 1.0807x over previous
import jax
import jax.numpy as jnp
from jax.experimental import pallas as pl

S = 2048
DIM = 768
H = 12
DH = 64
BLK = 4
NB = S // BLK
WIN = 32
QC = 256
NQ = S // QC
GCOL = 128
SCALE = DH ** -0.5
F32 = jnp.float32
BF16 = jnp.bfloat16


def _prep_kernel(x_ref, x0_ref, lam_ref, w_ref, x1_ref, y_ref):
    lam0 = lam_ref[0, 0]
    lam1 = lam_ref[0, 1]
    x1 = lam0 * x_ref[...] + lam1 * x0_ref[...]
    x1_ref[...] = x1
    h = x1 * jax.lax.rsqrt(jnp.mean(x1 * x1, axis=-1, keepdims=True) + 1e-6)
    y = jnp.dot(h.astype(BF16), w_ref[...], preferred_element_type=F32)
    y_ref[:, : 3 * DIM] = y[:, : 3 * DIM]
    y_ref[:, 3 * DIM :] = jax.nn.sigmoid(y[:, 3 * DIM :])


def _ckv_kernel(kh_ref, vh_ref, wkc_ref, wvc_ref, kpe_ref, vpe_ref, ck_ref, cv_ref):
    pe_k = jnp.dot(kpe_ref[...], wkc_ref[...], preferred_element_type=F32)
    pe_v = jnp.dot(vpe_ref[...], wvc_ref[...], preferred_element_type=F32)
    ck_ref[0] = (jnp.dot(kh_ref[0], wkc_ref[...], preferred_element_type=F32) + pe_k).astype(BF16)
    cv_ref[0] = (jnp.dot(vh_ref[0], wvc_ref[...], preferred_element_type=F32) + pe_v).astype(BF16)


QR = 512
NCALL = S // QR
SB = QR + 64


def _attn_one_head(j, q, k, v, ck, cv, fm, g):
    W = (j + 1) * QR
    CB = W // BLK
    s0 = max(0, j * QR - 64)
    sim = jax.lax.dot_general(q, k, (((1,), (1,)), ((), ())),
                              preferred_element_type=F32) * SCALE

    sf = sim * fm + (fm - 1.0) * 1e9
    mf = jnp.max(sf, axis=-1, keepdims=True)
    pf = jnp.exp(sf - mf)
    f_out = jax.lax.dot_general(pf.astype(BF16), v, (((1,), (0,)), ((), ())),
                                preferred_element_type=F32) / jnp.sum(
        pf, axis=-1, keepdims=True
    )

    simc = jax.lax.dot_general(q, ck, (((1,), (1,)), ((), ())),
                               preferred_element_type=F32) * SCALE
    rowc = j * QR + jax.lax.broadcasted_iota(jnp.int32, (QR, CB), 0)
    colc = jax.lax.broadcasted_iota(jnp.int32, (QR, CB), 1)
    cmask = ((colc + 1) * BLK - 1) <= rowc
    sc = jnp.where(cmask, simc, -1e9)
    mc = jnp.maximum(jnp.max(sc, axis=-1, keepdims=True), 0.0)
    pc = jnp.where(cmask, jnp.exp(sc - mc), 0.0)
    den = jnp.sum(pc, axis=-1, keepdims=True) + jnp.exp(-mc)
    c_out = jax.lax.dot_general(pc.astype(BF16), cv, (((1,), (0,)), ((), ())),
                                preferred_element_type=F32) / den

    sbw = min(SB, W - s0)
    kb = k[s0 : s0 + sbw, :]
    vb = v[s0 : s0 + sbw, :]
    sims = jax.lax.dot_general(q, kb, (((1,), (1,)), ((), ())),
                               preferred_element_type=F32) * SCALE
    rows = j * QR + jax.lax.broadcasted_iota(jnp.int32, (QR, sbw), 0)
    cols = s0 + jax.lax.broadcasted_iota(jnp.int32, (QR, sbw), 1)
    sm = (cols <= rows) & ((rows - cols) < WIN)
    ss = jnp.where(sm, sims, -1e9)
    ms = jnp.max(ss, axis=-1, keepdims=True)
    ps = jnp.where(sm, jnp.exp(ss - ms), 0.0)
    s_out = jax.lax.dot_general(ps.astype(BF16), vb, (((1,), (0,)), ((), ())),
                                preferred_element_type=F32) / jnp.sum(
        ps, axis=-1, keepdims=True
    )

    return g[:, 0:1] * c_out + g[:, 1:2] * f_out + g[:, 2:3] * s_out


def _attn_band_kernel(j, q_ref, k_ref, v_ref, ck_ref, cv_ref, fm_ref, g_ref,
                      _acc_ref, out_ref):
    fm = fm_ref[...].astype(F32)
    outs = []
    for t in range(2):
        outs.append(_attn_one_head(
            j, q_ref[t], k_ref[t], v_ref[t], ck_ref[t], cv_ref[t], fm,
            g_ref[t]))
    out_ref[...] = jnp.concatenate(outs, axis=-1).astype(BF16)


def _mlp_kernel(attn_ref, x1_ref, wo_ref, wfc_ref, wproj_ref, y_ref):
    x2 = x1_ref[...] + jnp.dot(attn_ref[...], wo_ref[...], preferred_element_type=F32)
    h2 = x2 * jax.lax.rsqrt(jnp.mean(x2 * x2, axis=-1, keepdims=True) + 1e-6)
    u = jnp.dot(h2.astype(BF16), wfc_ref[...], preferred_element_type=F32)
    u = jnp.square(jnp.maximum(u, 0.0))
    y_ref[...] = x2 + jnp.dot(u.astype(BF16), wproj_ref[...], preferred_element_type=F32)


def kernel(x, ve, x0, lambdas, Wq, Wk, Wv, Wo, k_pe, v_pe, Wkc, Wvc, Wg, Wfc, Wproj,
           sliding_window_flex_mask, fine_selection_flex_mask):
    del ve, sliding_window_flex_mask
    x2d = x[0]
    x02d = x0[0]
    w_all = jnp.concatenate(
        [Wq, Wk, Wv, jnp.pad(Wg, ((0, 0), (0, GCOL - 3 * H)))], axis=1
    ).astype(BF16)
    lam2 = lambdas.reshape(1, 2)

    x1, y = pl.pallas_call(
        _prep_kernel,
        grid=(NQ,),
        in_specs=[
            pl.BlockSpec((QC, DIM), lambda i: (i, 0)),
            pl.BlockSpec((QC, DIM), lambda i: (i, 0)),
            pl.BlockSpec((1, 2), lambda i: (0, 0)),
            pl.BlockSpec((DIM, 3 * DIM + GCOL), lambda i: (0, 0)),
        ],
        out_specs=[
            pl.BlockSpec((QC, DIM), lambda i: (i, 0)),
            pl.BlockSpec((QC, 3 * DIM + GCOL), lambda i: (i, 0)),
        ],
        out_shape=[
            jax.ShapeDtypeStruct((S, DIM), F32),
            jax.ShapeDtypeStruct((S, 3 * DIM + GCOL), F32),
        ],
    )(x2d, x02d, lam2, w_all)

    qkv = y[:, : 3 * DIM].reshape(S, 3, H, DH).transpose(1, 2, 0, 3)
    qkv16 = qkv.astype(BF16)
    q_hm = qkv16[0]
    k_hm = qkv16[1]
    v_hm = qkv16[2]
    g_hm = y[:, 3 * DIM : 3 * DIM + 3 * H].reshape(S, H, 3).transpose(1, 0, 2)
    kh = k_hm.reshape(H, NB, BLK * DH)
    vh = v_hm.reshape(H, NB, BLK * DH)

    ck, cv = pl.pallas_call(
        _ckv_kernel,
        grid=(H,),
        in_specs=[
            pl.BlockSpec((1, NB, BLK * DH), lambda h: (h, 0, 0)),
            pl.BlockSpec((1, NB, BLK * DH), lambda h: (h, 0, 0)),
            pl.BlockSpec((BLK * DH, DH), lambda h: (0, 0)),
            pl.BlockSpec((BLK * DH, DH), lambda h: (0, 0)),
            pl.BlockSpec((1, BLK * DH), lambda h: (0, 0)),
            pl.BlockSpec((1, BLK * DH), lambda h: (0, 0)),
        ],
        out_specs=[
            pl.BlockSpec((1, NB, DH), lambda h: (h, 0, 0)),
            pl.BlockSpec((1, NB, DH), lambda h: (h, 0, 0)),
        ],
        out_shape=[
            jax.ShapeDtypeStruct((H, NB, DH), BF16),
            jax.ShapeDtypeStruct((H, NB, DH), BF16),
        ],
    )(kh, vh, Wkc.astype(BF16), Wvc.astype(BF16),
      k_pe.reshape(1, BLK * DH).astype(BF16), v_pe.reshape(1, BLK * DH).astype(BF16))

    fm8 = fine_selection_flex_mask.astype(jnp.int8)

    import functools
    attn2d = jnp.zeros((S, H * DH), BF16)
    for j in range(NCALL):
        W = (j + 1) * QR
        CB = W // BLK
        attn2d = pl.pallas_call(
            functools.partial(_attn_band_kernel, j),
            grid=(H // 2,),
            in_specs=[
                pl.BlockSpec((2, QR, DH), lambda m, j=j: (m, j, 0)),
                pl.BlockSpec((2, W, DH), lambda m: (m, 0, 0)),
                pl.BlockSpec((2, W, DH), lambda m: (m, 0, 0)),
                pl.BlockSpec((2, CB, DH), lambda m: (m, 0, 0)),
                pl.BlockSpec((2, CB, DH), lambda m: (m, 0, 0)),
                pl.BlockSpec((QR, W), lambda m, j=j: (j, 0)),
                pl.BlockSpec((2, QR, 3), lambda m, j=j: (m, j, 0)),
                pl.BlockSpec((QR, 2 * DH), lambda m, j=j: (j, m)),
            ],
            out_specs=pl.BlockSpec((QR, 2 * DH), lambda m, j=j: (j, m)),
            out_shape=jax.ShapeDtypeStruct((S, H * DH), BF16),
            input_output_aliases={7: 0},
        )(q_hm, k_hm, v_hm, ck, cv, fm8, g_hm, attn2d)

    out = pl.pallas_call(
        _mlp_kernel,
        grid=(NQ,),
        in_specs=[
            pl.BlockSpec((QC, H * DH), lambda i: (i, 0)),
            pl.BlockSpec((QC, DIM), lambda i: (i, 0)),
            pl.BlockSpec((H * DH, DIM), lambda i: (0, 0)),
            pl.BlockSpec((DIM, 4 * DIM), lambda i: (0, 0)),
            pl.BlockSpec((4 * DIM, DIM), lambda i: (0, 0)),
        ],
        out_specs=pl.BlockSpec((QC, DIM), lambda i: (i, 0)),
        out_shape=jax.ShapeDtypeStruct((S, DIM), F32),
    )(attn2d, x1, Wo.astype(BF16), Wfc.astype(BF16), Wproj.astype(BF16))

    return out[None]


# speedup vs baseline: 2.0775x; 1.0279x over previous
import jax
import jax.numpy as jnp
from jax.experimental import pallas as pl

S = 2048
DIM = 768
H = 12
DH = 64
BLK = 4
NB = S // BLK
WIN = 32
QC = 256
NQ = S // QC
GCOL = 128
SCALE = DH ** -0.5
F32 = jnp.float32
BF16 = jnp.bfloat16


def _prep_kernel(x_ref, x0_ref, lam_ref, w_ref, x1_ref, y_ref):
    lam0 = lam_ref[0, 0]
    lam1 = lam_ref[0, 1]
    x1 = lam0 * x_ref[...] + lam1 * x0_ref[...]
    x1_ref[...] = x1
    h = x1 * jax.lax.rsqrt(jnp.mean(x1 * x1, axis=-1, keepdims=True) + 1e-6)
    y = jnp.dot(h.astype(BF16), w_ref[...], preferred_element_type=F32)
    y_ref[:, : 3 * DIM] = y[:, : 3 * DIM]
    y_ref[:, 3 * DIM :] = jax.nn.sigmoid(y[:, 3 * DIM :])


def _ckv_kernel(kh_ref, vh_ref, wkc_ref, wvc_ref, kpe_ref, vpe_ref, ck_ref, cv_ref):
    pe_k = jnp.dot(kpe_ref[...], wkc_ref[...], preferred_element_type=F32)
    pe_v = jnp.dot(vpe_ref[...], wvc_ref[...], preferred_element_type=F32)
    ck_ref[0] = (jnp.dot(kh_ref[0], wkc_ref[...], preferred_element_type=F32) + pe_k).astype(BF16)
    cv_ref[0] = (jnp.dot(vh_ref[0], wvc_ref[...], preferred_element_type=F32) + pe_v).astype(BF16)


QR = 512
NCALL = S // QR
SB = QR + 64


def _attn_one_head(j, q, k, v, ck, cv, fm, g):
    W = (j + 1) * QR
    CB = W // BLK
    s0 = max(0, j * QR - 64)
    sim = jax.lax.dot_general(q, k, (((1,), (1,)), ((), ())),
                              preferred_element_type=F32) * SCALE

    sf = sim * fm + (fm - 1.0) * 1e9
    mf = jnp.max(sf, axis=-1, keepdims=True)
    pf = jnp.exp(sf - mf)
    f_out = jax.lax.dot_general(pf.astype(BF16), v, (((1,), (0,)), ((), ())),
                                preferred_element_type=F32) / jnp.sum(
        pf, axis=-1, keepdims=True
    )

    simc = jax.lax.dot_general(q, ck, (((1,), (1,)), ((), ())),
                               preferred_element_type=F32) * SCALE
    rowc = j * QR + jax.lax.broadcasted_iota(jnp.int32, (QR, CB), 0)
    colc = jax.lax.broadcasted_iota(jnp.int32, (QR, CB), 1)
    cmask = ((colc + 1) * BLK - 1) <= rowc
    sc = jnp.where(cmask, simc, -1e9)
    mc = jnp.maximum(jnp.max(sc, axis=-1, keepdims=True), 0.0)
    pc = jnp.where(cmask, jnp.exp(sc - mc), 0.0)
    den = jnp.sum(pc, axis=-1, keepdims=True) + jnp.exp(-mc)
    c_out = jax.lax.dot_general(pc.astype(BF16), cv, (((1,), (0,)), ((), ())),
                                preferred_element_type=F32) / den

    sbw = min(SB, W - s0)
    kb = k[s0 : s0 + sbw, :]
    vb = v[s0 : s0 + sbw, :]
    sims = jax.lax.dot_general(q, kb, (((1,), (1,)), ((), ())),
                               preferred_element_type=F32) * SCALE
    rows = j * QR + jax.lax.broadcasted_iota(jnp.int32, (QR, sbw), 0)
    cols = s0 + jax.lax.broadcasted_iota(jnp.int32, (QR, sbw), 1)
    sm = (cols <= rows) & ((rows - cols) < WIN)
    ss = jnp.where(sm, sims, -1e9)
    ms = jnp.max(ss, axis=-1, keepdims=True)
    ps = jnp.where(sm, jnp.exp(ss - ms), 0.0)
    s_out = jax.lax.dot_general(ps.astype(BF16), vb, (((1,), (0,)), ((), ())),
                                preferred_element_type=F32) / jnp.sum(
        ps, axis=-1, keepdims=True
    )

    return g[:, 0:1] * c_out + g[:, 1:2] * f_out + g[:, 2:3] * s_out


def _attn_band_kernel(q_ref, k_ref, v_ref, ck_ref, cv_ref, fm_ref, g_ref,
                      _acc_ref, out_ref):
    j = pl.program_id(0)

    def mk(jj):
        W = (jj + 1) * QR

        def br():
            fm = fm_ref[:, :W].astype(F32)
            outs = []
            for t in range(2):
                outs.append(_attn_one_head(
                    jj, q_ref[t], k_ref[t, :W, :], v_ref[t, :W, :],
                    ck_ref[t, : W // BLK, :], cv_ref[t, : W // BLK, :],
                    fm, g_ref[t]))
            out_ref[...] = jnp.concatenate(outs, axis=-1).astype(BF16)
        return br

    jax.lax.switch(j, [mk(0), mk(1), mk(2), mk(3)])


def _mlp_kernel(attn_ref, x1_ref, wo_ref, wfc_ref, wproj_ref, y_ref):
    x2 = x1_ref[...] + jnp.dot(attn_ref[...], wo_ref[...], preferred_element_type=F32)
    h2 = x2 * jax.lax.rsqrt(jnp.mean(x2 * x2, axis=-1, keepdims=True) + 1e-6)
    u = jnp.dot(h2.astype(BF16), wfc_ref[...], preferred_element_type=F32)
    u = jnp.square(jnp.maximum(u, 0.0))
    y_ref[...] = x2 + jnp.dot(u.astype(BF16), wproj_ref[...], preferred_element_type=F32)


def kernel(x, ve, x0, lambdas, Wq, Wk, Wv, Wo, k_pe, v_pe, Wkc, Wvc, Wg, Wfc, Wproj,
           sliding_window_flex_mask, fine_selection_flex_mask):
    del ve, sliding_window_flex_mask
    x2d = x[0]
    x02d = x0[0]
    w_all = jnp.concatenate(
        [Wq, Wk, Wv, jnp.pad(Wg, ((0, 0), (0, GCOL - 3 * H)))], axis=1
    ).astype(BF16)
    lam2 = lambdas.reshape(1, 2)

    x1, y = pl.pallas_call(
        _prep_kernel,
        grid=(NQ,),
        in_specs=[
            pl.BlockSpec((QC, DIM), lambda i: (i, 0)),
            pl.BlockSpec((QC, DIM), lambda i: (i, 0)),
            pl.BlockSpec((1, 2), lambda i: (0, 0)),
            pl.BlockSpec((DIM, 3 * DIM + GCOL), lambda i: (0, 0)),
        ],
        out_specs=[
            pl.BlockSpec((QC, DIM), lambda i: (i, 0)),
            pl.BlockSpec((QC, 3 * DIM + GCOL), lambda i: (i, 0)),
        ],
        out_shape=[
            jax.ShapeDtypeStruct((S, DIM), F32),
            jax.ShapeDtypeStruct((S, 3 * DIM + GCOL), F32),
        ],
    )(x2d, x02d, lam2, w_all)

    qkv = y[:, : 3 * DIM].reshape(S, 3, H, DH).transpose(1, 2, 0, 3)
    qkv16 = qkv.astype(BF16)
    q_hm = qkv16[0]
    k_hm = qkv16[1]
    v_hm = qkv16[2]
    g_hm = y[:, 3 * DIM : 3 * DIM + 3 * H].reshape(S, H, 3).transpose(1, 0, 2)
    kh = k_hm.reshape(H, NB, BLK * DH)
    vh = v_hm.reshape(H, NB, BLK * DH)

    ck, cv = pl.pallas_call(
        _ckv_kernel,
        grid=(H,),
        in_specs=[
            pl.BlockSpec((1, NB, BLK * DH), lambda h: (h, 0, 0)),
            pl.BlockSpec((1, NB, BLK * DH), lambda h: (h, 0, 0)),
            pl.BlockSpec((BLK * DH, DH), lambda h: (0, 0)),
            pl.BlockSpec((BLK * DH, DH), lambda h: (0, 0)),
            pl.BlockSpec((1, BLK * DH), lambda h: (0, 0)),
            pl.BlockSpec((1, BLK * DH), lambda h: (0, 0)),
        ],
        out_specs=[
            pl.BlockSpec((1, NB, DH), lambda h: (h, 0, 0)),
            pl.BlockSpec((1, NB, DH), lambda h: (h, 0, 0)),
        ],
        out_shape=[
            jax.ShapeDtypeStruct((H, NB, DH), BF16),
            jax.ShapeDtypeStruct((H, NB, DH), BF16),
        ],
    )(kh, vh, Wkc.astype(BF16), Wvc.astype(BF16),
      k_pe.reshape(1, BLK * DH).astype(BF16), v_pe.reshape(1, BLK * DH).astype(BF16))

    fm8 = fine_selection_flex_mask.astype(jnp.int8)

    attn2d = pl.pallas_call(
        _attn_band_kernel,
        grid=(NCALL, H // 2),
        in_specs=[
            pl.BlockSpec((2, QR, DH), lambda j, m: (m, j, 0)),
            pl.BlockSpec((2, S, DH), lambda j, m: (m, 0, 0)),
            pl.BlockSpec((2, S, DH), lambda j, m: (m, 0, 0)),
            pl.BlockSpec((2, NB, DH), lambda j, m: (m, 0, 0)),
            pl.BlockSpec((2, NB, DH), lambda j, m: (m, 0, 0)),
            pl.BlockSpec((QR, S), lambda j, m: (j, 0)),
            pl.BlockSpec((2, QR, 3), lambda j, m: (m, j, 0)),
            pl.BlockSpec((QR, 2 * DH), lambda j, m: (j, m)),
        ],
        out_specs=pl.BlockSpec((QR, 2 * DH), lambda j, m: (j, m)),
        out_shape=jax.ShapeDtypeStruct((S, H * DH), BF16),
        input_output_aliases={7: 0},
    )(q_hm, k_hm, v_hm, ck, cv, fm8, g_hm, jnp.zeros((S, H * DH), BF16))

    out = pl.pallas_call(
        _mlp_kernel,
        grid=(NQ,),
        in_specs=[
            pl.BlockSpec((QC, H * DH), lambda i: (i, 0)),
            pl.BlockSpec((QC, DIM), lambda i: (i, 0)),
            pl.BlockSpec((H * DH, DIM), lambda i: (0, 0)),
            pl.BlockSpec((DIM, 4 * DIM), lambda i: (0, 0)),
            pl.BlockSpec((4 * DIM, DIM), lambda i: (0, 0)),
        ],
        out_specs=pl.BlockSpec((QC, DIM), lambda i: (i, 0)),
        out_shape=jax.ShapeDtypeStruct((S, DIM), F32),
    )(attn2d, x1, Wo.astype(BF16), Wfc.astype(BF16), Wproj.astype(BF16))

    return out[None]


# speedup vs baseline: 2.6131x; 1.2578x over previous
import jax
import jax.numpy as jnp
from jax.experimental import pallas as pl

S = 2048
DIM = 768
H = 12
DH = 64
BLK = 4
NB = S // BLK
WIN = 32
QC = 256
NQ = S // QC
GCOL = 128
SCALE = DH ** -0.5
F32 = jnp.float32
BF16 = jnp.bfloat16


def _prep_kernel(x_ref, x0_ref, lam_ref, w_ref, x1_ref, q_ref, k_ref, v_ref,
                 g_ref):
    lam0 = lam_ref[0, 0]
    lam1 = lam_ref[0, 1]
    x1 = lam0 * x_ref[...] + lam1 * x0_ref[...]
    x1_ref[...] = x1
    h = x1 * jax.lax.rsqrt(jnp.mean(x1 * x1, axis=-1, keepdims=True) + 1e-6)
    y = jnp.dot(h.astype(BF16), w_ref[...], preferred_element_type=F32)
    for hh in range(H):
        q_ref[hh] = y[:, hh * DH : (hh + 1) * DH].astype(BF16)
        k_ref[hh] = y[:, DIM + hh * DH : DIM + (hh + 1) * DH].astype(BF16)
        v_ref[hh] = y[:, 2 * DIM + hh * DH : 2 * DIM + (hh + 1) * DH].astype(BF16)
        g_ref[hh] = jax.nn.sigmoid(y[:, 3 * DIM + 3 * hh : 3 * DIM + 3 * hh + 4])


def _ckv_kernel(kh_ref, vh_ref, wkc_ref, wvc_ref, kpe_ref, vpe_ref, ck_ref, cv_ref):
    pe_k = jnp.dot(kpe_ref[...], wkc_ref[...], preferred_element_type=F32)
    pe_v = jnp.dot(vpe_ref[...], wvc_ref[...], preferred_element_type=F32)
    ck_ref[0] = (jnp.dot(kh_ref[0], wkc_ref[...], preferred_element_type=F32) + pe_k).astype(BF16)
    cv_ref[0] = (jnp.dot(vh_ref[0], wvc_ref[...], preferred_element_type=F32) + pe_v).astype(BF16)


QR = 512
NCALL = S // QR
SB = QR + 64


def _attn_one_head(j, q, k, v, ck, cv, fm, g):
    W = (j + 1) * QR
    CB = W // BLK
    s0 = max(0, j * QR - 64)
    sim = jax.lax.dot_general(q, k, (((1,), (1,)), ((), ())),
                              preferred_element_type=F32) * SCALE

    sf = sim * fm + (fm - 1.0) * 1e9
    mf = jnp.max(sf, axis=-1, keepdims=True)
    pf = jnp.exp(sf - mf)
    f_out = jax.lax.dot_general(pf.astype(BF16), v, (((1,), (0,)), ((), ())),
                                preferred_element_type=F32) / jnp.sum(
        pf, axis=-1, keepdims=True
    )

    simc = jax.lax.dot_general(q, ck, (((1,), (1,)), ((), ())),
                               preferred_element_type=F32) * SCALE
    rowc = j * QR + jax.lax.broadcasted_iota(jnp.int32, (QR, CB), 0)
    colc = jax.lax.broadcasted_iota(jnp.int32, (QR, CB), 1)
    cmask = ((colc + 1) * BLK - 1) <= rowc
    sc = jnp.where(cmask, simc, -1e9)
    mc = jnp.maximum(jnp.max(sc, axis=-1, keepdims=True), 0.0)
    pc = jnp.where(cmask, jnp.exp(sc - mc), 0.0)
    den = jnp.sum(pc, axis=-1, keepdims=True) + jnp.exp(-mc)
    c_out = jax.lax.dot_general(pc.astype(BF16), cv, (((1,), (0,)), ((), ())),
                                preferred_element_type=F32) / den

    sbw = min(SB, W - s0)
    kb = k[s0 : s0 + sbw, :]
    vb = v[s0 : s0 + sbw, :]
    sims = jax.lax.dot_general(q, kb, (((1,), (1,)), ((), ())),
                               preferred_element_type=F32) * SCALE
    rows = j * QR + jax.lax.broadcasted_iota(jnp.int32, (QR, sbw), 0)
    cols = s0 + jax.lax.broadcasted_iota(jnp.int32, (QR, sbw), 1)
    sm = (cols <= rows) & ((rows - cols) < WIN)
    ss = jnp.where(sm, sims, -1e9)
    ms = jnp.max(ss, axis=-1, keepdims=True)
    ps = jnp.where(sm, jnp.exp(ss - ms), 0.0)
    s_out = jax.lax.dot_general(ps.astype(BF16), vb, (((1,), (0,)), ((), ())),
                                preferred_element_type=F32) / jnp.sum(
        ps, axis=-1, keepdims=True
    )

    return g[:, 0:1] * c_out + g[:, 1:2] * f_out + g[:, 2:3] * s_out


def _attn_band_kernel(q_ref, k_ref, v_ref, ck_ref, cv_ref, fm_ref, g_ref,
                      _acc_ref, out_ref):
    j = pl.program_id(0)

    def mk(jj):
        W = (jj + 1) * QR

        def br():
            fm = fm_ref[:, :W].astype(F32)
            outs = []
            for t in range(2):
                outs.append(_attn_one_head(
                    jj, q_ref[t], k_ref[t, :W, :], v_ref[t, :W, :],
                    ck_ref[t, : W // BLK, :], cv_ref[t, : W // BLK, :],
                    fm, g_ref[t]))
            out_ref[...] = jnp.concatenate(outs, axis=-1).astype(BF16)
        return br

    jax.lax.switch(j, [mk(0), mk(1), mk(2), mk(3)])


def _mlp_kernel(attn_ref, x1_ref, wo_ref, wfc_ref, wproj_ref, y_ref):
    x2 = x1_ref[...] + jnp.dot(attn_ref[...], wo_ref[...], preferred_element_type=F32)
    h2 = x2 * jax.lax.rsqrt(jnp.mean(x2 * x2, axis=-1, keepdims=True) + 1e-6)
    u = jnp.dot(h2.astype(BF16), wfc_ref[...], preferred_element_type=F32)
    u = jnp.square(jnp.maximum(u, 0.0))
    y_ref[...] = x2 + jnp.dot(u.astype(BF16), wproj_ref[...], preferred_element_type=F32)


def kernel(x, ve, x0, lambdas, Wq, Wk, Wv, Wo, k_pe, v_pe, Wkc, Wvc, Wg, Wfc, Wproj,
           sliding_window_flex_mask, fine_selection_flex_mask):
    del ve, sliding_window_flex_mask
    x2d = x[0]
    x02d = x0[0]
    w_all = jnp.concatenate(
        [Wq, Wk, Wv, jnp.pad(Wg, ((0, 0), (0, GCOL - 3 * H)))], axis=1
    ).astype(BF16)
    lam2 = lambdas.reshape(1, 2)

    x1, q_hm, k_hm, v_hm, g_hm = pl.pallas_call(
        _prep_kernel,
        grid=(NQ,),
        in_specs=[
            pl.BlockSpec((QC, DIM), lambda i: (i, 0)),
            pl.BlockSpec((QC, DIM), lambda i: (i, 0)),
            pl.BlockSpec((1, 2), lambda i: (0, 0)),
            pl.BlockSpec((DIM, 3 * DIM + GCOL), lambda i: (0, 0)),
        ],
        out_specs=[
            pl.BlockSpec((QC, DIM), lambda i: (i, 0)),
            pl.BlockSpec((H, QC, DH), lambda i: (0, i, 0)),
            pl.BlockSpec((H, QC, DH), lambda i: (0, i, 0)),
            pl.BlockSpec((H, QC, DH), lambda i: (0, i, 0)),
            pl.BlockSpec((H, QC, 4), lambda i: (0, i, 0)),
        ],
        out_shape=[
            jax.ShapeDtypeStruct((S, DIM), F32),
            jax.ShapeDtypeStruct((H, S, DH), BF16),
            jax.ShapeDtypeStruct((H, S, DH), BF16),
            jax.ShapeDtypeStruct((H, S, DH), BF16),
            jax.ShapeDtypeStruct((H, S, 4), F32),
        ],
    )(x2d, x02d, lam2, w_all)

    kh = k_hm.reshape(H, NB, BLK * DH)
    vh = v_hm.reshape(H, NB, BLK * DH)

    ck, cv = pl.pallas_call(
        _ckv_kernel,
        grid=(H,),
        in_specs=[
            pl.BlockSpec((1, NB, BLK * DH), lambda h: (h, 0, 0)),
            pl.BlockSpec((1, NB, BLK * DH), lambda h: (h, 0, 0)),
            pl.BlockSpec((BLK * DH, DH), lambda h: (0, 0)),
            pl.BlockSpec((BLK * DH, DH), lambda h: (0, 0)),
            pl.BlockSpec((1, BLK * DH), lambda h: (0, 0)),
            pl.BlockSpec((1, BLK * DH), lambda h: (0, 0)),
        ],
        out_specs=[
            pl.BlockSpec((1, NB, DH), lambda h: (h, 0, 0)),
            pl.BlockSpec((1, NB, DH), lambda h: (h, 0, 0)),
        ],
        out_shape=[
            jax.ShapeDtypeStruct((H, NB, DH), BF16),
            jax.ShapeDtypeStruct((H, NB, DH), BF16),
        ],
    )(kh, vh, Wkc.astype(BF16), Wvc.astype(BF16),
      k_pe.reshape(1, BLK * DH).astype(BF16), v_pe.reshape(1, BLK * DH).astype(BF16))

    fm8 = fine_selection_flex_mask.astype(jnp.int8)

    attn2d = pl.pallas_call(
        _attn_band_kernel,
        grid=(NCALL, H // 2),
        in_specs=[
            pl.BlockSpec((2, QR, DH), lambda j, m: (m, j, 0)),
            pl.BlockSpec((2, S, DH), lambda j, m: (m, 0, 0)),
            pl.BlockSpec((2, S, DH), lambda j, m: (m, 0, 0)),
            pl.BlockSpec((2, NB, DH), lambda j, m: (m, 0, 0)),
            pl.BlockSpec((2, NB, DH), lambda j, m: (m, 0, 0)),
            pl.BlockSpec((QR, S), lambda j, m: (j, 0)),
            pl.BlockSpec((2, QR, 4), lambda j, m: (m, j, 0)),
            pl.BlockSpec((QR, 2 * DH), lambda j, m: (j, m)),
        ],
        out_specs=pl.BlockSpec((QR, 2 * DH), lambda j, m: (j, m)),
        out_shape=jax.ShapeDtypeStruct((S, H * DH), BF16),
        input_output_aliases={7: 0},
    )(q_hm, k_hm, v_hm, ck, cv, fm8, g_hm, jnp.zeros((S, H * DH), BF16))

    out = pl.pallas_call(
        _mlp_kernel,
        grid=(NQ,),
        in_specs=[
            pl.BlockSpec((QC, H * DH), lambda i: (i, 0)),
            pl.BlockSpec((QC, DIM), lambda i: (i, 0)),
            pl.BlockSpec((H * DH, DIM), lambda i: (0, 0)),
            pl.BlockSpec((DIM, 4 * DIM), lambda i: (0, 0)),
            pl.BlockSpec((4 * DIM, DIM), lambda i: (0, 0)),
        ],
        out_specs=pl.BlockSpec((QC, DIM), lambda i: (i, 0)),
        out_shape=jax.ShapeDtypeStruct((S, DIM), F32),
    )(attn2d, x1, Wo.astype(BF16), Wfc.astype(BF16), Wproj.astype(BF16))

    return out[None]
